# Initial kernel scaffold; baseline (speedup 1.0000x reference)
import jax
import jax.numpy as jnp
from jax.experimental import pallas as pl


def kernel(x, W_router, W_gate, W_up, W_down):
    raise NotImplementedError("write your pallas kernel here")


# speedup vs baseline: 1.4731x; 1.4731x over previous
import functools

import jax
import jax.numpy as jnp
from jax.experimental import pallas as pl
from jax.experimental.pallas import tpu as pltpu

E = 8
TOP_K = 2
ALPHA = 0.01
D_MODEL = 1024
D_FF = 2816
N_TOK = 2048
F_BLK = 256
N_F = D_FF // F_BLK


def _router_body(x_ref, wr_ref, wdense_ref, counts_ref, aux_ref):
    x = x_ref[...]
    wr = wr_ref[...]
    logits = jax.lax.dot_general(
        x, wr, (((1,), (1,)), ((), ())),
        preferred_element_type=jnp.float32)
    m = jnp.max(logits, axis=-1, keepdims=True)
    ex = jnp.exp(logits - m)
    s = jnp.sum(ex, axis=-1, keepdims=True)
    probs = ex / s

    e_iota = jax.lax.broadcasted_iota(jnp.int32, (N_TOK, E), 1)
    v1 = jnp.max(probs, axis=-1, keepdims=True)
    i1 = jnp.min(jnp.where(probs == v1, e_iota, E), axis=-1, keepdims=True)
    probs_m = jnp.where(e_iota == i1, -1.0, probs)
    v2 = jnp.max(probs_m, axis=-1, keepdims=True)
    i2 = jnp.min(jnp.where(probs_m == v2, e_iota, E), axis=-1, keepdims=True)

    tsum = v1 + v2
    w1 = v1 / tsum
    w2 = v2 / tsum
    sel1 = (e_iota == i1)
    sel2 = (e_iota == i2)
    wdense_ref[...] = jnp.where(sel1, w1, 0.0) + jnp.where(sel2, w2, 0.0)

    counts = jnp.sum(sel1.astype(jnp.float32) + sel2.astype(jnp.float32),
                     axis=0, keepdims=True)
    counts_ref[...] = counts.astype(jnp.int32)
    p_mean = jnp.mean(probs, axis=0, keepdims=True)
    f_i = counts / float(N_TOK * TOP_K)
    aux_ref[...] = (ALPHA * E) * jnp.sum(f_i * p_mean, keepdims=True).reshape(1, 1)


def _router_call(xf, w_router):
    return pl.pallas_call(
        _router_body,
        out_shape=(
            jax.ShapeDtypeStruct((N_TOK, E), jnp.float32),
            jax.ShapeDtypeStruct((1, E), jnp.int32),
            jax.ShapeDtypeStruct((1, 1), jnp.float32),
        ),
    )(xf, w_router)


def _ffn_body(w_ref, xb_ref, wg_ref, wu_ref, wd_ref, out_ref):
    e = pl.program_id(0)
    f = pl.program_id(1)

    @pl.when((e == 0) & (f == 0))
    def _init():
        out_ref[...] = jnp.zeros_like(out_ref)

    xb = xb_ref[...]
    wg = wg_ref[0].astype(jnp.bfloat16)
    wu = wu_ref[0].astype(jnp.bfloat16)
    wd = wd_ref[0].astype(jnp.bfloat16)
    g = jnp.dot(xb, wg, preferred_element_type=jnp.float32)
    u = jnp.dot(xb, wu, preferred_element_type=jnp.float32)
    h = g * jax.nn.sigmoid(g) * u
    w_all = w_ref[...]
    e_iota = jax.lax.broadcasted_iota(jnp.int32, (N_TOK, E), 1)
    w_col = jnp.sum(jnp.where(e_iota == e, w_all, 0.0), axis=1, keepdims=True)
    h = h * w_col
    out_ref[...] += jnp.dot(h.astype(jnp.bfloat16), wd,
                            preferred_element_type=jnp.float32)


def _ffn_call(w_dense, x_bf16, w_gate, w_up, w_down):
    return pl.pallas_call(
        _ffn_body,
        grid=(E, N_F),
        in_specs=[
            pl.BlockSpec((N_TOK, E), lambda e, f: (0, 0)),
            pl.BlockSpec((N_TOK, D_MODEL), lambda e, f: (0, 0)),
            pl.BlockSpec((1, D_MODEL, F_BLK), lambda e, f: (e, 0, f)),
            pl.BlockSpec((1, D_MODEL, F_BLK), lambda e, f: (e, 0, f)),
            pl.BlockSpec((1, F_BLK, D_MODEL), lambda e, f: (e, f, 0)),
        ],
        out_specs=pl.BlockSpec((N_TOK, D_MODEL), lambda e, f: (0, 0)),
        out_shape=jax.ShapeDtypeStruct((N_TOK, D_MODEL), jnp.float32),
        compiler_params=pltpu.CompilerParams(
            dimension_semantics=("arbitrary", "arbitrary")),
    )(w_dense, x_bf16, w_gate, w_up, w_down)


def kernel(x, W_router, W_gate, W_up, W_down):
    B, S, D = x.shape
    xf = x.reshape(S, D)
    w_dense, counts, aux = _router_call(xf, W_router)
    out = _ffn_call(w_dense, xf.astype(jnp.bfloat16), W_gate, W_up, W_down)
    return out.reshape(B, S, D), aux.reshape(())
